# baseline (speedup 1.0000x reference)
import functools
import math

import jax
import jax.numpy as jnp
from jax import lax
from jax.experimental import pallas as pl
from jax.experimental.pallas import tpu as pltpu

_B = 65536
_D = 16
_RD = 8

_ATAN_COEF = (
    0.9999999750196067,
    -0.3333319678737739,
    0.19996762077387733,
    -0.14250134989068346,
    0.10891953621690719,
    -0.08252553645313367,
    0.05567456985706047,
    -0.029126335253611,
    0.009906937955111169,
    -0.0015853064817422272,
)

_DEG = float(180.0 / math.pi)
_PI = float(math.pi)
_HALF_PI = float(math.pi / 2.0)


def _zone_from_xy(x, y):
    ax = jnp.abs(x)
    ay = jnp.abs(y)
    den = jnp.maximum(ax, ay)
    num = jnp.minimum(ax, ay)
    safe_den = jnp.maximum(den, jnp.float32(1e-30))
    r = num / safe_den
    r2 = r * r
    acc = jnp.float32(_ATAN_COEF[-1])
    for c in _ATAN_COEF[-2::-1]:
        acc = acc * r2 + jnp.float32(c)
    a = acc * r
    a = jnp.where(ay > ax, jnp.float32(_HALF_PI) - a, a)
    a = jnp.where(x < 0.0, jnp.float32(_PI) - a, a)
    deg = a * jnp.float32(_DEG)
    deg = jnp.where(y < 0.0, jnp.float32(360.0) - deg, deg)
    k = deg.astype(jnp.int32)
    pcs = k + 90
    pcs = jnp.where(pcs >= 360, pcs - 360, pcs)
    t = ((pcs.astype(jnp.float32) + jnp.float32(11.5))
         * jnp.float32(1.0 / 22.0)).astype(jnp.int32)
    return jnp.bitwise_and(t, 7)


def _body(z_ref, dir_ref, out_ref):
    z = z_ref[...]
    x = z[:, 3] - z[:, 1]
    y = z[:, 2] - z[:, 4]
    z0 = z[:, 0]
    zone = _zone_from_xy(x, y)
    d = dir_ref[...]
    oh = lax.broadcasted_iota(jnp.int32, d.shape, 1) == zone[:, None]
    val = jnp.sum(jnp.where(oh, d, jnp.float32(0.0)), axis=1)
    out_ref[...] = jnp.where(z0 == 0.0, jnp.float32(0.0), val)


@jax.jit
def kernel(z_1, dir):
    rows = 4096
    grid = _B // rows
    return pl.pallas_call(
        _body,
        grid=(grid,),
        in_specs=[
            pl.BlockSpec((rows, _D), lambda i: (i, 0)),
            pl.BlockSpec((rows, _RD), lambda i: (i, 0)),
        ],
        out_specs=pl.BlockSpec((rows,), lambda i: (i,)),
        out_shape=jax.ShapeDtypeStruct((_B,), jnp.float32),
        compiler_params=pltpu.CompilerParams(
            dimension_semantics=("arbitrary",)),
    )(z_1, dir)


# speedup vs baseline: 8.1648x; 8.1648x over previous
import math

import jax
import jax.numpy as jnp
from jax import lax
from jax.experimental import pallas as pl
from jax.experimental.pallas import tpu as pltpu

_B = 65536
_W = 128
_H = _B // _W

_ATAN_COEF = (
    0.9999999750196067,
    -0.3333319678737739,
    0.19996762077387733,
    -0.14250134989068346,
    0.10891953621690719,
    -0.08252553645313367,
    0.05567456985706047,
    -0.029126335253611,
    0.009906937955111169,
    -0.0015853064817422272,
)

_DEG = float(180.0 / math.pi)
_PI = float(math.pi)
_HALF_PI = float(math.pi / 2.0)


def _zone_from_xy(x, y):
    ax = jnp.abs(x)
    ay = jnp.abs(y)
    den = jnp.maximum(ax, ay)
    num = jnp.minimum(ax, ay)
    safe_den = jnp.maximum(den, jnp.float32(1e-30))
    r = num / safe_den
    r2 = r * r
    acc = jnp.float32(_ATAN_COEF[-1])
    for c in _ATAN_COEF[-2::-1]:
        acc = acc * r2 + jnp.float32(c)
    a = acc * r
    a = jnp.where(ay > ax, jnp.float32(_HALF_PI) - a, a)
    a = jnp.where(x < 0.0, jnp.float32(_PI) - a, a)
    deg = a * jnp.float32(_DEG)
    deg = jnp.where(y < 0.0, jnp.float32(360.0) - deg, deg)
    k = deg.astype(jnp.int32)
    pcs = k + 90
    pcs = jnp.where(pcs >= 360, pcs - 360, pcs)
    t = ((pcs.astype(jnp.float32) + jnp.float32(11.5))
         * jnp.float32(1.0 / 22.0)).astype(jnp.int32)
    return jnp.bitwise_and(t, 7)


def _body(z0_ref, c1_ref, c2_ref, c3_ref, c4_ref,
          d0_ref, d1_ref, d2_ref, d3_ref, d4_ref, d5_ref, d6_ref, d7_ref,
          out_ref):
    x = c3_ref[...] - c1_ref[...]
    y = c2_ref[...] - c4_ref[...]
    zone = _zone_from_xy(x, y)
    d_refs = (d0_ref, d1_ref, d2_ref, d3_ref, d4_ref, d5_ref, d6_ref, d7_ref)
    val = jnp.where(zone == 0, d_refs[0][...], jnp.float32(0.0))
    for c in range(1, 8):
        val = jnp.where(zone == c, d_refs[c][...], val)
    out_ref[...] = jnp.where(z0_ref[...] == 0.0, jnp.float32(0.0), val)


@jax.jit
def kernel(z_1, dir):
    cols = [z_1[:, j].reshape(_H, _W) for j in (0, 1, 2, 3, 4)]
    dcols = [dir[:, j].reshape(_H, _W) for j in range(8)]
    rows = 64
    grid = _H // rows
    spec = pl.BlockSpec((rows, _W), lambda i: (i, 0))
    out = pl.pallas_call(
        _body,
        grid=(grid,),
        in_specs=[spec] * 13,
        out_specs=spec,
        out_shape=jax.ShapeDtypeStruct((_H, _W), jnp.float32),
        compiler_params=pltpu.CompilerParams(
            dimension_semantics=("arbitrary",)),
    )(*cols, *dcols)
    return out.reshape(_B)


# speedup vs baseline: 15.1779x; 1.8589x over previous
import math

import jax
import jax.numpy as jnp
from jax import lax
from jax.experimental import pallas as pl
from jax.experimental.pallas import tpu as pltpu

_B = 65536
_W = 128
_H = _B // _W

_ATAN_COEF = (
    0.9999999750196067,
    -0.3333319678737739,
    0.19996762077387733,
    -0.14250134989068346,
    0.10891953621690719,
    -0.08252553645313367,
    0.05567456985706047,
    -0.029126335253611,
    0.009906937955111169,
    -0.0015853064817422272,
)

_DEG = float(180.0 / math.pi)
_PI = float(math.pi)
_HALF_PI = float(math.pi / 2.0)


def _zone_from_xy(x, y):
    ax = jnp.abs(x)
    ay = jnp.abs(y)
    den = jnp.maximum(ax, ay)
    num = jnp.minimum(ax, ay)
    safe_den = jnp.maximum(den, jnp.float32(1e-30))
    r = num / safe_den
    r2 = r * r
    acc = jnp.float32(_ATAN_COEF[-1])
    for c in _ATAN_COEF[-2::-1]:
        acc = acc * r2 + jnp.float32(c)
    a = acc * r
    a = jnp.where(ay > ax, jnp.float32(_HALF_PI) - a, a)
    a = jnp.where(x < 0.0, jnp.float32(_PI) - a, a)
    deg = a * jnp.float32(_DEG)
    deg = jnp.where(y < 0.0, jnp.float32(360.0) - deg, deg)
    k = deg.astype(jnp.int32)
    pcs = k + 90
    pcs = jnp.where(pcs >= 360, pcs - 360, pcs)
    t = ((pcs.astype(jnp.float32) + jnp.float32(11.5))
         * jnp.float32(1.0 / 22.0)).astype(jnp.int32)
    return jnp.bitwise_and(t, 7)


def _body(z0_ref, c1_ref, c2_ref, c3_ref, c4_ref, d_ref, out_ref):
    x = c3_ref[0] - c1_ref[0]
    y = c2_ref[0] - c4_ref[0]
    zone = _zone_from_xy(x, y)
    val = jnp.where(zone == 0, d_ref[0], jnp.float32(0.0))
    for c in range(1, 8):
        val = jnp.where(zone == c, d_ref[c], val)
    out_ref[...] = jnp.where(z0_ref[0] == 0.0, jnp.float32(0.0), val)


@jax.jit
def kernel(z_1, dir):
    zT = z_1.T.reshape(16, _H, _W)
    dT = dir.T.reshape(8, _H, _W)
    rows = 64
    grid = _H // rows

    def col(j):
        return pl.BlockSpec((1, rows, _W), lambda i, j=j: (j, i, 0))

    out = pl.pallas_call(
        _body,
        grid=(grid,),
        in_specs=[col(0), col(1), col(2), col(3), col(4),
                  pl.BlockSpec((8, rows, _W), lambda i: (0, i, 0))],
        out_specs=pl.BlockSpec((rows, _W), lambda i: (i, 0)),
        out_shape=jax.ShapeDtypeStruct((_H, _W), jnp.float32),
        compiler_params=pltpu.CompilerParams(
            dimension_semantics=("arbitrary",)),
    )(zT, zT, zT, zT, zT, dT)
    return out.reshape(_B)


# speedup vs baseline: 20.1060x; 1.3247x over previous
import math

import jax
import jax.numpy as jnp
from jax import lax
from jax.experimental import pallas as pl
from jax.experimental.pallas import tpu as pltpu

_B = 65536
_W = 128
_H = _B // _W

_ATAN_COEF = (
    0.9999999750196067,
    -0.3333319678737739,
    0.19996762077387733,
    -0.14250134989068346,
    0.10891953621690719,
    -0.08252553645313367,
    0.05567456985706047,
    -0.029126335253611,
    0.009906937955111169,
    -0.0015853064817422272,
)

_DEG = float(180.0 / math.pi)
_PI = float(math.pi)
_HALF_PI = float(math.pi / 2.0)


def _zone_from_xy(x, y):
    ax = jnp.abs(x)
    ay = jnp.abs(y)
    den = jnp.maximum(ax, ay)
    num = jnp.minimum(ax, ay)
    safe_den = jnp.maximum(den, jnp.float32(1e-30))
    r = num / safe_den
    r2 = r * r
    acc = jnp.float32(_ATAN_COEF[-1])
    for c in _ATAN_COEF[-2::-1]:
        acc = acc * r2 + jnp.float32(c)
    a = acc * r
    a = jnp.where(ay > ax, jnp.float32(_HALF_PI) - a, a)
    a = jnp.where(x < 0.0, jnp.float32(_PI) - a, a)
    deg = a * jnp.float32(_DEG)
    deg = jnp.where(y < 0.0, jnp.float32(360.0) - deg, deg)
    k = deg.astype(jnp.int32)
    pcs = k + 90
    pcs = jnp.where(pcs >= 360, pcs - 360, pcs)
    t = ((pcs.astype(jnp.float32) + jnp.float32(11.5))
         * jnp.float32(1.0 / 22.0)).astype(jnp.int32)
    return jnp.bitwise_and(t, 7)


def _body(z_ref, d_ref, out_ref):
    zb = z_ref[0]
    db = d_ref[0]
    x = zb[:, 3, :] - zb[:, 1, :]
    y = zb[:, 2, :] - zb[:, 4, :]
    zone = _zone_from_xy(x, y)
    val = jnp.where(zone == 0, db[:, 0, :], jnp.float32(0.0))
    for c in range(1, 8):
        val = jnp.where(zone == c, db[:, c, :], val)
    out_ref[...] = jnp.where(zb[:, 0, :] == 0.0, jnp.float32(0.0), val)


@jax.jit
def kernel(z_1, dir):
    z4 = z_1.reshape(_H, _W, 2, 8).transpose(2, 0, 3, 1)
    d4 = dir.reshape(_H, _W, 1, 8).transpose(2, 0, 3, 1)
    rows = 64
    grid = _H // rows
    spec = pl.BlockSpec((1, rows, 8, _W), lambda i: (0, i, 0, 0))
    out = pl.pallas_call(
        _body,
        grid=(grid,),
        in_specs=[spec, spec],
        out_specs=pl.BlockSpec((rows, _W), lambda i: (i, 0)),
        out_shape=jax.ShapeDtypeStruct((_H, _W), jnp.float32),
        compiler_params=pltpu.CompilerParams(
            dimension_semantics=("arbitrary",)),
    )(z4, d4)
    return out.reshape(_B)


# speedup vs baseline: 29.2428x; 1.4544x over previous
import math

import jax
import jax.numpy as jnp
from jax import lax
from jax.experimental import pallas as pl
from jax.experimental.pallas import tpu as pltpu

_B = 65536
_W = 128
_H = _B // _W

_ATAN_COEF = (
    0.9999999750196067,
    -0.3333319678737739,
    0.19996762077387733,
    -0.14250134989068346,
    0.10891953621690719,
    -0.08252553645313367,
    0.05567456985706047,
    -0.029126335253611,
    0.009906937955111169,
    -0.0015853064817422272,
)

_DEG = float(180.0 / math.pi)
_PI = float(math.pi)
_HALF_PI = float(math.pi / 2.0)


def _zone_from_xy(x, y):
    ax = jnp.abs(x)
    ay = jnp.abs(y)
    den = jnp.maximum(ax, ay)
    num = jnp.minimum(ax, ay)
    safe_den = jnp.maximum(den, jnp.float32(1e-30))
    r = num / safe_den
    r2 = r * r
    acc = jnp.float32(_ATAN_COEF[-1])
    for c in _ATAN_COEF[-2::-1]:
        acc = acc * r2 + jnp.float32(c)
    a = acc * r
    a = jnp.where(ay > ax, jnp.float32(_HALF_PI) - a, a)
    a = jnp.where(x < 0.0, jnp.float32(_PI) - a, a)
    deg = a * jnp.float32(_DEG)
    deg = jnp.where(y < 0.0, jnp.float32(360.0) - deg, deg)
    k = deg.astype(jnp.int32)
    pcs = k + 90
    pcs = jnp.where(pcs >= 360, pcs - 360, pcs)
    t = ((pcs.astype(jnp.float32) + jnp.float32(11.5))
         * jnp.float32(1.0 / 22.0)).astype(jnp.int32)
    return jnp.bitwise_and(t, 7)


def _body(z_ref, d_ref, out_ref):
    zt = jnp.transpose(z_ref[0], (1, 0, 2))
    dt = jnp.transpose(d_ref[0], (1, 0, 2))
    x = zt[3] - zt[1]
    y = zt[2] - zt[4]
    zone = _zone_from_xy(x, y)
    val = jnp.where(zone == 0, dt[0], jnp.float32(0.0))
    for c in range(1, 8):
        val = jnp.where(zone == c, dt[c], val)
    out_ref[...] = jnp.where(zt[0] == 0.0, jnp.float32(0.0), val)


@jax.jit
def kernel(z_1, dir):
    z4 = z_1.reshape(_H, _W, 2, 8).transpose(2, 0, 3, 1)
    d4 = dir.reshape(_H, _W, 1, 8).transpose(2, 0, 3, 1)
    rows = 64
    grid = _H // rows
    spec = pl.BlockSpec((1, rows, 8, _W), lambda i: (0, i, 0, 0))
    out = pl.pallas_call(
        _body,
        grid=(grid,),
        in_specs=[spec, spec],
        out_specs=pl.BlockSpec((rows, _W), lambda i: (i, 0)),
        out_shape=jax.ShapeDtypeStruct((_H, _W), jnp.float32),
        compiler_params=pltpu.CompilerParams(
            dimension_semantics=("arbitrary",)),
    )(z4, d4)
    return out.reshape(_B)


# speedup vs baseline: 41.6103x; 1.4229x over previous
import math

import jax
import jax.numpy as jnp
from jax import lax
from jax.experimental import pallas as pl
from jax.experimental.pallas import tpu as pltpu

_B = 65536
_W = 128
_H = _B // _W

_ATAN_COEF = (
    0.9999999750196067,
    -0.3333319678737739,
    0.19996762077387733,
    -0.14250134989068346,
    0.10891953621690719,
    -0.08252553645313367,
    0.05567456985706047,
    -0.029126335253611,
    0.009906937955111169,
    -0.0015853064817422272,
)

_DEG = float(180.0 / math.pi)
_PI = float(math.pi)
_HALF_PI = float(math.pi / 2.0)


def _zone_from_xy(x, y):
    ax = jnp.abs(x)
    ay = jnp.abs(y)
    den = jnp.maximum(ax, ay)
    num = jnp.minimum(ax, ay)
    safe_den = jnp.maximum(den, jnp.float32(1e-30))
    r = num / safe_den
    r2 = r * r
    acc = jnp.float32(_ATAN_COEF[-1])
    for c in _ATAN_COEF[-2::-1]:
        acc = acc * r2 + jnp.float32(c)
    a = acc * r
    a = jnp.where(ay > ax, jnp.float32(_HALF_PI) - a, a)
    a = jnp.where(x < 0.0, jnp.float32(_PI) - a, a)
    deg = a * jnp.float32(_DEG)
    deg = jnp.where(y < 0.0, jnp.float32(360.0) - deg, deg)
    k = deg.astype(jnp.int32)
    pcs = k + 90
    pcs = jnp.where(pcs >= 360, pcs - 360, pcs)
    t = ((pcs.astype(jnp.float32) + jnp.float32(11.5))
         * jnp.float32(1.0 / 22.0)).astype(jnp.int32)
    return jnp.bitwise_and(t, 7)


def _body(z_ref, d_ref, out_ref):
    zt = jnp.transpose(z_ref[0], (1, 0, 2))
    dt = jnp.transpose(d_ref[0], (1, 0, 2))
    x = zt[3] - zt[1]
    y = zt[2] - zt[4]
    zone = _zone_from_xy(x, y)
    val = jnp.where(zone == 0, dt[0], jnp.float32(0.0))
    for c in range(1, 8):
        val = jnp.where(zone == c, dt[c], val)
    out_ref[...] = jnp.where(zt[0] == 0.0, jnp.float32(0.0), val)


@jax.jit
def kernel(z_1, dir):
    z4 = z_1.reshape(_H, _W, 2, 8).transpose(2, 0, 3, 1)
    d4 = dir.reshape(_H, _W, 1, 8).transpose(2, 0, 3, 1)
    rows = 128
    grid = _H // rows
    spec = pl.BlockSpec((1, rows, 8, _W), lambda i: (0, i, 0, 0))
    out = pl.pallas_call(
        _body,
        grid=(grid,),
        in_specs=[spec, spec],
        out_specs=pl.BlockSpec((rows, _W), lambda i: (i, 0)),
        out_shape=jax.ShapeDtypeStruct((_H, _W), jnp.float32),
        compiler_params=pltpu.CompilerParams(
            dimension_semantics=("arbitrary",)),
    )(z4, d4)
    return out.reshape(_B)


# speedup vs baseline: 52.4498x; 1.2605x over previous
import math

import jax
import jax.numpy as jnp
from jax import lax
from jax.experimental import pallas as pl
from jax.experimental.pallas import tpu as pltpu

_B = 65536
_W = 128
_H = _B // _W

_ATAN_COEF = (
    0.9999999750196067,
    -0.3333319678737739,
    0.19996762077387733,
    -0.14250134989068346,
    0.10891953621690719,
    -0.08252553645313367,
    0.05567456985706047,
    -0.029126335253611,
    0.009906937955111169,
    -0.0015853064817422272,
)

_DEG = float(180.0 / math.pi)
_PI = float(math.pi)
_HALF_PI = float(math.pi / 2.0)


def _zone_from_xy(x, y):
    ax = jnp.abs(x)
    ay = jnp.abs(y)
    den = jnp.maximum(ax, ay)
    num = jnp.minimum(ax, ay)
    safe_den = jnp.maximum(den, jnp.float32(1e-30))
    r = num / safe_den
    r2 = r * r
    acc = jnp.float32(_ATAN_COEF[-1])
    for c in _ATAN_COEF[-2::-1]:
        acc = acc * r2 + jnp.float32(c)
    a = acc * r
    a = jnp.where(ay > ax, jnp.float32(_HALF_PI) - a, a)
    a = jnp.where(x < 0.0, jnp.float32(_PI) - a, a)
    deg = a * jnp.float32(_DEG)
    deg = jnp.where(y < 0.0, jnp.float32(360.0) - deg, deg)
    k = deg.astype(jnp.int32)
    pcs = k + 90
    pcs = jnp.where(pcs >= 360, pcs - 360, pcs)
    t = ((pcs.astype(jnp.float32) + jnp.float32(11.5))
         * jnp.float32(1.0 / 22.0)).astype(jnp.int32)
    return jnp.bitwise_and(t, 7)


def _body(z_ref, d_ref, out_ref):
    zt = jnp.transpose(z_ref[0], (1, 0, 2))
    dt = jnp.transpose(d_ref[0], (1, 0, 2))
    x = zt[3] - zt[1]
    y = zt[2] - zt[4]
    zone = _zone_from_xy(x, y)
    val = jnp.where(zone == 0, dt[0], jnp.float32(0.0))
    for c in range(1, 8):
        val = jnp.where(zone == c, dt[c], val)
    out_ref[...] = jnp.where(zt[0] == 0.0, jnp.float32(0.0), val)


@jax.jit
def kernel(z_1, dir):
    z4 = z_1.reshape(_H, _W, 2, 8).transpose(2, 0, 3, 1)
    d4 = dir.reshape(_H, _W, 1, 8).transpose(2, 0, 3, 1)
    rows = 256
    grid = _H // rows
    spec = pl.BlockSpec((1, rows, 8, _W), lambda i: (0, i, 0, 0))
    out = pl.pallas_call(
        _body,
        grid=(grid,),
        in_specs=[spec, spec],
        out_specs=pl.BlockSpec((rows, _W), lambda i: (i, 0)),
        out_shape=jax.ShapeDtypeStruct((_H, _W), jnp.float32),
        compiler_params=pltpu.CompilerParams(
            dimension_semantics=("arbitrary",)),
    )(z4, d4)
    return out.reshape(_B)
